# baseline (speedup 1.0000x reference)
import functools
import math

import jax
import jax.numpy as jnp
from jax import lax
from jax.experimental import pallas as pl
from jax.experimental.pallas import tpu as pltpu
from jax.experimental.pallas import tpu_sc as plsc

_N = 10000
_E = 320000
_D = 128
_HID = 64
_NB = 8
_NL = 3
_SH = 15

_NC = 2
_NS = 16
_K = 80
_EPT = _E // _NS
_CHUNKS = _EPT // _K
_EPW = _E // (_NC * _NS)
_GCHUNKS = _EPW // _K
_RPT = _N // _NS


def _swish(x):
    return x / (1.0 + jnp.exp(-x))


def _pos_gather_body(pos16, send, recv, out_d, idx, bufs, bufr, sem):
    c = lax.axis_index("c")
    s = lax.axis_index("s")
    w = s * _NC + c

    def chunk(i, carry):
        base = w * _EPW + i * _K
        pltpu.sync_copy(send.at[pl.ds(base, _K)], idx)
        pltpu.async_copy(pos16.at[idx], bufs, sem).wait()
        pltpu.sync_copy(recv.at[pl.ds(base, _K)], idx)
        pltpu.async_copy(pos16.at[idx], bufr, sem).wait()

        def sub_row(k, cr):
            bufr[k, :] = bufr[k, :] - bufs[k, :]
            return cr

        lax.fori_loop(0, _K, sub_row, 0)
        pltpu.sync_copy(bufr, out_d.at[pl.ds(base, _K)])
        return carry

    lax.fori_loop(0, _GCHUNKS, chunk, 0)


def _msg_body(hu, coeff, send, recv, zrows, out, sidx, ridx, cbuf, rows, acc, sem):
    c = lax.axis_index("c")
    s = lax.axis_index("s")
    pltpu.sync_copy(zrows, acc.at[pl.ds(s * _RPT, _RPT)])
    plsc.subcore_barrier()

    def chunk(i, carry):
        base = s * _EPT + i * _K
        pltpu.sync_copy(send.at[pl.ds(base, _K)], sidx)
        pltpu.sync_copy(recv.at[pl.ds(base, _K)], ridx)
        pltpu.sync_copy(coeff.at[c, pl.ds(base, _K)], cbuf)
        pltpu.async_copy(hu.at[sidx], rows, sem).wait()

        def mul_row(k, cr):
            for j in range(_D // 16):
                sl = pl.ds(j * 16, 16)
                rows[k, sl] = rows[k, sl] * cbuf[k, sl]
            return cr

        lax.fori_loop(0, _K, mul_row, 0)
        pltpu.sync_copy(rows, acc.at[ridx], add=True)
        return carry

    lax.fori_loop(0, _CHUNKS, chunk, 0)
    plsc.subcore_barrier()
    pltpu.sync_copy(acc.at[pl.ds(s * _RPT, _RPT)],
                    out.at[c, pl.ds(s * _RPT, _RPT)])


@functools.cache
def _sc_kernels():
    mesh = plsc.VectorSubcoreMesh(core_axis_name="c", subcore_axis_name="s")
    pos_gather = pl.kernel(
        _pos_gather_body,
        mesh=mesh,
        compiler_params=pltpu.CompilerParams(use_tc_tiling_on_sc=False),
        out_type=jax.ShapeDtypeStruct((_E, 16), jnp.float32),
        scratch_types=[
            pltpu.VMEM((_K,), jnp.int32),
            pltpu.VMEM((_K, 16), jnp.float32),
            pltpu.VMEM((_K, 16), jnp.float32),
            pltpu.SemaphoreType.DMA,
        ],
    )
    msg_pass = pl.kernel(
        _msg_body,
        mesh=mesh,
        compiler_params=pltpu.CompilerParams(use_tc_tiling_on_sc=False),
        out_type=jax.ShapeDtypeStruct((2, _N, _D), jnp.float32),
        scratch_types=[
            pltpu.VMEM((_K,), jnp.int32),
            pltpu.VMEM((_K,), jnp.int32),
            pltpu.VMEM((_K, _D), jnp.float32),
            pltpu.VMEM((_K, _D), jnp.float32),
            pltpu.VMEM_SHARED((_N, _D), jnp.float32),
            pltpu.SemaphoreType.DMA,
        ],
    )
    return pos_gather, msg_pass


_EPAD = 327680
_EP = _EPAD // 128
_BR = 32
_BV = 512


def _vtr_body(d_ref, ident_ref, vt_ref):
    vt_ref[...] = lax.dot_general(d_ref[...], ident_ref[...],
                                  (((0,), (0,)), ((), ())),
                                  precision=lax.Precision.HIGHEST,
                                  preferred_element_type=jnp.float32)


_vtr_call = pl.pallas_call(
    _vtr_body,
    grid=(_E // _BV,),
    in_specs=[
        pl.BlockSpec((_BV, 16), lambda e: (e, 0)),
        pl.BlockSpec((_BV, _BV), lambda e: (0, 0)),
    ],
    out_specs=pl.BlockSpec((16, _BV), lambda e: (0, e)),
    out_shape=jax.ShapeDtypeStruct((16, _EPAD), jnp.float32),
)


def _geom_body(vx_ref, vy_ref, vz_ref, rb_ref, sh_ref):
    x = vx_ref[0]
    y = vy_ref[0]
    z = vz_ref[0]
    sq = x * x + y * y + z * z
    is0 = sq == 0.0
    r = jnp.sqrt(jnp.where(is0, 1.0, sq))
    r = jnp.where(is0, 0.0, r)
    inv = 1.0 / jnp.where(is0, 1.0, r)
    ux = x * inv
    uy = y * inv
    uz = z * inv
    s3 = math.sqrt(3.0)
    s15 = math.sqrt(15.0)
    sh_ref[0] = s3 * ux
    sh_ref[1] = s3 * uy
    sh_ref[2] = s3 * uz
    sh_ref[3] = s15 * ux * uy
    sh_ref[4] = s15 * uy * uz
    sh_ref[5] = (math.sqrt(5.0) / 2.0) * (3.0 * uz * uz - 1.0)
    sh_ref[6] = s15 * ux * uz
    sh_ref[7] = (s15 / 2.0) * (ux * ux - uy * uy)
    sh_ref[8] = (math.sqrt(70.0) / 4.0) * uy * (3.0 * ux * ux - uy * uy)
    sh_ref[9] = math.sqrt(105.0) * ux * uy * uz
    sh_ref[10] = (math.sqrt(42.0) / 4.0) * uy * (5.0 * uz * uz - 1.0)
    sh_ref[11] = (math.sqrt(7.0) / 2.0) * uz * (5.0 * uz * uz - 3.0)
    sh_ref[12] = (math.sqrt(42.0) / 4.0) * ux * (5.0 * uz * uz - 1.0)
    sh_ref[13] = (math.sqrt(105.0) / 2.0) * uz * (ux * ux - uy * uy)
    sh_ref[14] = (math.sqrt(70.0) / 4.0) * ux * (ux * ux - 3.0 * uy * uy)
    r2 = r * r
    r5 = r2 * r2 * r
    poly = 1.0 - 21.0 * r5 + 35.0 * r5 * r - 15.0 * r5 * r2
    env = jnp.where(r < 1.0, poly, 0.0)
    fac = math.sqrt(2.0) * jnp.where(is0, 0.0, env * inv)
    for k in range(_NB):
        rb_ref[k] = jnp.sin((math.pi * (k + 1)) * r) * fac


_geom_call = pl.pallas_call(
    _geom_body,
    grid=(_EP // _BR,),
    in_specs=[
        pl.BlockSpec((1, _BR, 128), lambda i: (0, i, 0)),
        pl.BlockSpec((1, _BR, 128), lambda i: (1, i, 0)),
        pl.BlockSpec((1, _BR, 128), lambda i: (2, i, 0)),
    ],
    out_specs=[
        pl.BlockSpec((_NB, _BR, 128), lambda i: (0, i, 0)),
        pl.BlockSpec((_SH, _BR, 128), lambda i: (0, i, 0)),
    ],
    out_shape=[
        jax.ShapeDtypeStruct((_NB, _EP, 128), jnp.float32),
        jax.ShapeDtypeStruct((_SH, _EP, 128), jnp.float32),
    ],
)

_BE = 4096


def _coeff_body(rb_ref, sh_ref, m1t_ref, m2t_ref, m3_ref, wsh_ref, out_ref):
    t = _swish(jnp.dot(m1t_ref[...], rb_ref[...],
                       preferred_element_type=jnp.float32))
    t = _swish(jnp.dot(m2t_ref[...], t, preferred_element_type=jnp.float32))
    mix = lax.dot_general(t, m3_ref[...], (((0,), (0,)), ((), ())),
                          preferred_element_type=jnp.float32)
    wsh = lax.dot_general(sh_ref[...], wsh_ref[...], (((0,), (0,)), ((), ())),
                          preferred_element_type=jnp.float32)
    out_ref[0] = mix[:, :_D]
    out_ref[1] = wsh * mix[:, _D:]


_coeff_call = pl.pallas_call(
    _coeff_body,
    grid=(_EPAD // _BE,),
    in_specs=[
        pl.BlockSpec((_NB, _BE), lambda e: (0, e)),
        pl.BlockSpec((_SH, _BE), lambda e: (0, e)),
        pl.BlockSpec((_HID, _NB), lambda e: (0, 0)),
        pl.BlockSpec((_HID, _HID), lambda e: (0, 0)),
        pl.BlockSpec((_HID, 2 * _D), lambda e: (0, 0)),
        pl.BlockSpec((_SH, _D), lambda e: (0, 0)),
    ],
    out_specs=pl.BlockSpec((2, _BE, _D), lambda e: (0, e, 0)),
    out_shape=jax.ShapeDtypeStruct((2, _EPAD, _D), jnp.float32),
)

_BN = 2000


def _node_body(h_ref, w_ref, sc_ref, hu_ref):
    hw = jnp.dot(h_ref[...], w_ref[...], preferred_element_type=jnp.float32)
    sc_ref[...] = hw[:, :_D]
    hu_ref[...] = hw[:, _D:]


_node_call = pl.pallas_call(
    _node_body,
    grid=(_N // _BN,),
    in_specs=[
        pl.BlockSpec((_BN, _D), lambda i: (i, 0)),
        pl.BlockSpec((_D, 2 * _D), lambda i: (0, 0)),
    ],
    out_specs=[
        pl.BlockSpec((_BN, _D), lambda i: (i, 0)),
        pl.BlockSpec((_BN, _D), lambda i: (i, 0)),
    ],
    out_shape=[
        jax.ShapeDtypeStruct((_N, _D), jnp.float32),
        jax.ShapeDtypeStruct((_N, _D), jnp.float32),
    ],
)


def _down_body(a_ref, b_ref, sc_ref, wt_ref, wb_ref, h_ref):
    t = (jnp.dot(a_ref[0], wt_ref[...], preferred_element_type=jnp.float32)
         + jnp.dot(b_ref[0], wb_ref[...], preferred_element_type=jnp.float32)
         + sc_ref[...])
    h_ref[...] = _swish(t)


_down_call = pl.pallas_call(
    _down_body,
    grid=(_N // _BN,),
    in_specs=[
        pl.BlockSpec((1, _BN, _D), lambda i: (0, i, 0)),
        pl.BlockSpec((1, _BN, _D), lambda i: (1, i, 0)),
        pl.BlockSpec((_BN, _D), lambda i: (i, 0)),
        pl.BlockSpec((_D, _D), lambda i: (0, 0)),
        pl.BlockSpec((_D, _D), lambda i: (0, 0)),
    ],
    out_specs=pl.BlockSpec((_BN, _D), lambda i: (i, 0)),
    out_shape=jax.ShapeDtypeStruct((_N, _D), jnp.float32),
)


def kernel(positions, node_features, senders, receivers, W_sc, W_up, W_sh, W_down, M1, M2, M3):
    pos_gather, msg_pass = _sc_kernels()
    pos16 = jnp.pad(positions, ((0, 0), (0, 13)))
    dvec = pos_gather(pos16, senders, receivers)
    vt3 = _vtr_call(dvec, jnp.eye(_BV, dtype=jnp.float32)).reshape(16, _EP, 128)
    rb3, sh3 = _geom_call(vt3, vt3, vt3)
    rb2 = rb3.reshape(_NB, _EPAD)
    sh2 = sh3.reshape(_SH, _EPAD)
    coeffs = [
        _coeff_call(rb2, sh2, M1[l].T, M2[l].T, M3[l], W_sh[l])
        for l in range(_NL)
    ]
    zrows = jnp.zeros((_RPT, _D), jnp.float32)
    h = node_features
    for l in range(_NL):
        wn = jnp.concatenate([W_sc[l], W_up[l]], axis=1)
        sc, hu = _node_call(h, wn)
        agg = msg_pass(hu, coeffs[l], senders, receivers, zrows)
        h = _down_call(agg, agg, sc, W_down[l, :_D], W_down[l, _D:])
    return h[:, :3]


# speedup vs baseline: 1.4513x; 1.4513x over previous
import functools
import math

import jax
import jax.numpy as jnp
from jax import lax
from jax.experimental import pallas as pl
from jax.experimental.pallas import tpu as pltpu
from jax.experimental.pallas import tpu_sc as plsc

_N = 10000
_E = 320000
_D = 128
_HID = 64
_NB = 8
_NL = 3
_SH = 15

_NC = 2
_NS = 16
_K = 40
_EPT = _E // _NS
_CHUNKS = _EPT // _K
_EPW = _E // (_NC * _NS)
_GCHUNKS = _EPW // _K
_RPT = _N // _NS


def _swish(x):
    return x / (1.0 + jnp.exp(-x))


def _pos_gather_body(pos16, send, recv, out_d, idx, bufs, bufr, sem):
    c = lax.axis_index("c")
    s = lax.axis_index("s")
    w = s * _NC + c

    def chunk(i, carry):
        base = w * _EPW + i * _K
        pltpu.sync_copy(send.at[pl.ds(base, _K)], idx)
        pltpu.async_copy(pos16.at[idx], bufs, sem).wait()
        pltpu.sync_copy(recv.at[pl.ds(base, _K)], idx)
        pltpu.async_copy(pos16.at[idx], bufr, sem).wait()

        def sub_row(k, cr):
            bufr[k, :] = bufr[k, :] - bufs[k, :]
            return cr

        lax.fori_loop(0, _K, sub_row, 0)
        pltpu.sync_copy(bufr, out_d.at[pl.ds(base, _K)])
        return carry

    lax.fori_loop(0, _GCHUNKS, chunk, 0)


_NBUF = 4
_OUTER = _CHUNKS // _NBUF


def _msg_body(hu, coeff, send, recv, zrows, out, *scr):
    sidx = scr[0:_NBUF]
    ridx = scr[_NBUF:2 * _NBUF]
    cbuf = scr[2 * _NBUF:3 * _NBUF]
    rows = scr[3 * _NBUF:4 * _NBUF]
    acc = scr[4 * _NBUF]
    isem = scr[4 * _NBUF + 1:5 * _NBUF + 1]
    rsem = scr[5 * _NBUF + 1:6 * _NBUF + 1]
    gsem = scr[6 * _NBUF + 1:7 * _NBUF + 1]
    csem = scr[7 * _NBUF + 1:8 * _NBUF + 1]
    ssem = scr[8 * _NBUF + 1:9 * _NBUF + 1]
    c = lax.axis_index("c")
    s = lax.axis_index("s")
    pltpu.sync_copy(zrows, acc.at[pl.ds(s * _RPT, _RPT)])
    plsc.subcore_barrier()

    def outer(g, carry):
        base0 = s * _EPT + g * (_NBUF * _K)
        for b in range(_NBUF):
            @pl.when(g > 0)
            def _(b=b):
                pltpu.make_async_copy(rows[b], acc.at[ridx[b]], ssem[b]).wait()
            pltpu.async_copy(send.at[pl.ds(base0 + b * _K, _K)], sidx[b], isem[b])
            pltpu.async_copy(recv.at[pl.ds(base0 + b * _K, _K)], ridx[b], rsem[b])
        for b in range(_NBUF):
            pltpu.make_async_copy(send.at[pl.ds(base0 + b * _K, _K)], sidx[b], isem[b]).wait()
            pltpu.make_async_copy(recv.at[pl.ds(base0 + b * _K, _K)], ridx[b], rsem[b]).wait()
            pltpu.async_copy(hu.at[sidx[b]], rows[b], gsem[b])
            pltpu.async_copy(coeff.at[c, pl.ds(base0 + b * _K, _K)], cbuf[b], csem[b])
        for b in range(_NBUF):
            pltpu.make_async_copy(hu.at[sidx[b]], rows[b], gsem[b]).wait()
            pltpu.make_async_copy(coeff.at[c, pl.ds(base0 + b * _K, _K)], cbuf[b], csem[b]).wait()

            def mul_row(k, cr, b=b):
                for j in range(_D // 16):
                    sl = pl.ds(j * 16, 16)
                    rows[b][k, sl] = rows[b][k, sl] * cbuf[b][k, sl]
                return cr

            lax.fori_loop(0, _K, mul_row, 0)
            pltpu.async_copy(rows[b], acc.at[ridx[b]], ssem[b], add=True)
        return carry

    lax.fori_loop(0, _OUTER, outer, 0)
    for b in range(_NBUF):
        pltpu.make_async_copy(rows[b], acc.at[ridx[b]], ssem[b]).wait()
    plsc.subcore_barrier()
    pltpu.sync_copy(acc.at[pl.ds(s * _RPT, _RPT)],
                    out.at[c, pl.ds(s * _RPT, _RPT)])


@functools.cache
def _sc_kernels():
    mesh = plsc.VectorSubcoreMesh(core_axis_name="c", subcore_axis_name="s")
    pos_gather = pl.kernel(
        _pos_gather_body,
        mesh=mesh,
        compiler_params=pltpu.CompilerParams(use_tc_tiling_on_sc=False),
        out_type=jax.ShapeDtypeStruct((_E, 16), jnp.float32),
        scratch_types=[
            pltpu.VMEM((_K,), jnp.int32),
            pltpu.VMEM((_K, 16), jnp.float32),
            pltpu.VMEM((_K, 16), jnp.float32),
            pltpu.SemaphoreType.DMA,
        ],
    )
    msg_pass = pl.kernel(
        _msg_body,
        mesh=mesh,
        compiler_params=pltpu.CompilerParams(use_tc_tiling_on_sc=False),
        out_type=jax.ShapeDtypeStruct((2, _N, _D), jnp.float32),
        scratch_types=(
            [pltpu.VMEM((_K,), jnp.int32)] * (2 * _NBUF)
            + [pltpu.VMEM((_K, _D), jnp.float32)] * (2 * _NBUF)
            + [pltpu.VMEM_SHARED((_N, _D), jnp.float32)]
            + [pltpu.SemaphoreType.DMA] * (5 * _NBUF)
        ),
    )
    return pos_gather, msg_pass


_EPAD = 327680
_EP = _EPAD // 128
_BR = 32
_BV = 512


def _vtr_body(d_ref, ident_ref, vt_ref):
    vt_ref[...] = lax.dot_general(d_ref[...], ident_ref[...],
                                  (((0,), (0,)), ((), ())),
                                  precision=lax.Precision.HIGHEST,
                                  preferred_element_type=jnp.float32)


_vtr_call = pl.pallas_call(
    _vtr_body,
    grid=(_E // _BV,),
    in_specs=[
        pl.BlockSpec((_BV, 16), lambda e: (e, 0)),
        pl.BlockSpec((_BV, _BV), lambda e: (0, 0)),
    ],
    out_specs=pl.BlockSpec((16, _BV), lambda e: (0, e)),
    out_shape=jax.ShapeDtypeStruct((16, _EPAD), jnp.float32),
)


def _geom_body(vx_ref, vy_ref, vz_ref, rb_ref, sh_ref):
    x = vx_ref[0]
    y = vy_ref[0]
    z = vz_ref[0]
    sq = x * x + y * y + z * z
    is0 = sq == 0.0
    r = jnp.sqrt(jnp.where(is0, 1.0, sq))
    r = jnp.where(is0, 0.0, r)
    inv = 1.0 / jnp.where(is0, 1.0, r)
    ux = x * inv
    uy = y * inv
    uz = z * inv
    s3 = math.sqrt(3.0)
    s15 = math.sqrt(15.0)
    sh_ref[0] = s3 * ux
    sh_ref[1] = s3 * uy
    sh_ref[2] = s3 * uz
    sh_ref[3] = s15 * ux * uy
    sh_ref[4] = s15 * uy * uz
    sh_ref[5] = (math.sqrt(5.0) / 2.0) * (3.0 * uz * uz - 1.0)
    sh_ref[6] = s15 * ux * uz
    sh_ref[7] = (s15 / 2.0) * (ux * ux - uy * uy)
    sh_ref[8] = (math.sqrt(70.0) / 4.0) * uy * (3.0 * ux * ux - uy * uy)
    sh_ref[9] = math.sqrt(105.0) * ux * uy * uz
    sh_ref[10] = (math.sqrt(42.0) / 4.0) * uy * (5.0 * uz * uz - 1.0)
    sh_ref[11] = (math.sqrt(7.0) / 2.0) * uz * (5.0 * uz * uz - 3.0)
    sh_ref[12] = (math.sqrt(42.0) / 4.0) * ux * (5.0 * uz * uz - 1.0)
    sh_ref[13] = (math.sqrt(105.0) / 2.0) * uz * (ux * ux - uy * uy)
    sh_ref[14] = (math.sqrt(70.0) / 4.0) * ux * (ux * ux - 3.0 * uy * uy)
    r2 = r * r
    r5 = r2 * r2 * r
    poly = 1.0 - 21.0 * r5 + 35.0 * r5 * r - 15.0 * r5 * r2
    env = jnp.where(r < 1.0, poly, 0.0)
    fac = math.sqrt(2.0) * jnp.where(is0, 0.0, env * inv)
    for k in range(_NB):
        rb_ref[k] = jnp.sin((math.pi * (k + 1)) * r) * fac


_geom_call = pl.pallas_call(
    _geom_body,
    grid=(_EP // _BR,),
    in_specs=[
        pl.BlockSpec((1, _BR, 128), lambda i: (0, i, 0)),
        pl.BlockSpec((1, _BR, 128), lambda i: (1, i, 0)),
        pl.BlockSpec((1, _BR, 128), lambda i: (2, i, 0)),
    ],
    out_specs=[
        pl.BlockSpec((_NB, _BR, 128), lambda i: (0, i, 0)),
        pl.BlockSpec((_SH, _BR, 128), lambda i: (0, i, 0)),
    ],
    out_shape=[
        jax.ShapeDtypeStruct((_NB, _EP, 128), jnp.float32),
        jax.ShapeDtypeStruct((_SH, _EP, 128), jnp.float32),
    ],
)

_BE = 4096


def _coeff_body(rb_ref, sh_ref, m1t_ref, m2t_ref, m3_ref, wsh_ref, out_ref):
    t = _swish(jnp.dot(m1t_ref[...], rb_ref[...],
                       preferred_element_type=jnp.float32))
    t = _swish(jnp.dot(m2t_ref[...], t, preferred_element_type=jnp.float32))
    mix = lax.dot_general(t, m3_ref[...], (((0,), (0,)), ((), ())),
                          preferred_element_type=jnp.float32)
    wsh = lax.dot_general(sh_ref[...], wsh_ref[...], (((0,), (0,)), ((), ())),
                          preferred_element_type=jnp.float32)
    out_ref[0] = mix[:, :_D]
    out_ref[1] = wsh * mix[:, _D:]


_coeff_call = pl.pallas_call(
    _coeff_body,
    grid=(_EPAD // _BE,),
    in_specs=[
        pl.BlockSpec((_NB, _BE), lambda e: (0, e)),
        pl.BlockSpec((_SH, _BE), lambda e: (0, e)),
        pl.BlockSpec((_HID, _NB), lambda e: (0, 0)),
        pl.BlockSpec((_HID, _HID), lambda e: (0, 0)),
        pl.BlockSpec((_HID, 2 * _D), lambda e: (0, 0)),
        pl.BlockSpec((_SH, _D), lambda e: (0, 0)),
    ],
    out_specs=pl.BlockSpec((2, _BE, _D), lambda e: (0, e, 0)),
    out_shape=jax.ShapeDtypeStruct((2, _EPAD, _D), jnp.float32),
)

_BN = 2000


def _node_body(h_ref, w_ref, sc_ref, hu_ref):
    hw = jnp.dot(h_ref[...], w_ref[...], preferred_element_type=jnp.float32)
    sc_ref[...] = hw[:, :_D]
    hu_ref[...] = hw[:, _D:]


_node_call = pl.pallas_call(
    _node_body,
    grid=(_N // _BN,),
    in_specs=[
        pl.BlockSpec((_BN, _D), lambda i: (i, 0)),
        pl.BlockSpec((_D, 2 * _D), lambda i: (0, 0)),
    ],
    out_specs=[
        pl.BlockSpec((_BN, _D), lambda i: (i, 0)),
        pl.BlockSpec((_BN, _D), lambda i: (i, 0)),
    ],
    out_shape=[
        jax.ShapeDtypeStruct((_N, _D), jnp.float32),
        jax.ShapeDtypeStruct((_N, _D), jnp.float32),
    ],
)


def _down_body(a_ref, b_ref, sc_ref, wt_ref, wb_ref, h_ref):
    t = (jnp.dot(a_ref[0], wt_ref[...], preferred_element_type=jnp.float32)
         + jnp.dot(b_ref[0], wb_ref[...], preferred_element_type=jnp.float32)
         + sc_ref[...])
    h_ref[...] = _swish(t)


_down_call = pl.pallas_call(
    _down_body,
    grid=(_N // _BN,),
    in_specs=[
        pl.BlockSpec((1, _BN, _D), lambda i: (0, i, 0)),
        pl.BlockSpec((1, _BN, _D), lambda i: (1, i, 0)),
        pl.BlockSpec((_BN, _D), lambda i: (i, 0)),
        pl.BlockSpec((_D, _D), lambda i: (0, 0)),
        pl.BlockSpec((_D, _D), lambda i: (0, 0)),
    ],
    out_specs=pl.BlockSpec((_BN, _D), lambda i: (i, 0)),
    out_shape=jax.ShapeDtypeStruct((_N, _D), jnp.float32),
)


def kernel(positions, node_features, senders, receivers, W_sc, W_up, W_sh, W_down, M1, M2, M3):
    pos_gather, msg_pass = _sc_kernels()
    pos16 = jnp.pad(positions, ((0, 0), (0, 13)))
    dvec = pos_gather(pos16, senders, receivers)
    vt3 = _vtr_call(dvec, jnp.eye(_BV, dtype=jnp.float32)).reshape(16, _EP, 128)
    rb3, sh3 = _geom_call(vt3, vt3, vt3)
    rb2 = rb3.reshape(_NB, _EPAD)
    sh2 = sh3.reshape(_SH, _EPAD)
    coeffs = [
        _coeff_call(rb2, sh2, M1[l].T, M2[l].T, M3[l], W_sh[l])
        for l in range(_NL)
    ]
    zrows = jnp.zeros((_RPT, _D), jnp.float32)
    h = node_features
    for l in range(_NL):
        wn = jnp.concatenate([W_sc[l], W_up[l]], axis=1)
        sc, hu = _node_call(h, wn)
        agg = msg_pass(hu, coeffs[l], senders, receivers, zrows)
        h = _down_call(agg, agg, sc, W_down[l, :_D], W_down[l, _D:])
    return h[:, :3]


# speedup vs baseline: 1.7038x; 1.1740x over previous
import functools
import math

import jax
import jax.numpy as jnp
from jax import lax
from jax.experimental import pallas as pl
from jax.experimental.pallas import tpu as pltpu
from jax.experimental.pallas import tpu_sc as plsc

_N = 10000
_E = 320000
_D = 128
_HID = 64
_NB = 8
_NL = 3
_SH = 15

_NC = 2
_NS = 16
_K = 40
_EPT = _E // _NS
_CHUNKS = _EPT // _K
_EPW = _E // (_NC * _NS)
_GCHUNKS = _EPW // _K
_RPT = _N // _NS


def _swish(x):
    return x / (1.0 + jnp.exp(-x))


_GNBUF = 5
_GOUTER = _GCHUNKS // _GNBUF


def _pos_gather_body(pos16, send, recv, out_d, *scr):
    sidx = scr[0:_GNBUF]
    ridx = scr[_GNBUF:2 * _GNBUF]
    bufs = scr[2 * _GNBUF:3 * _GNBUF]
    bufr = scr[3 * _GNBUF:4 * _GNBUF]
    isem = scr[4 * _GNBUF:5 * _GNBUF]
    rsem = scr[5 * _GNBUF:6 * _GNBUF]
    gssem = scr[6 * _GNBUF:7 * _GNBUF]
    grsem = scr[7 * _GNBUF:8 * _GNBUF]
    osem = scr[8 * _GNBUF:9 * _GNBUF]
    c = lax.axis_index("c")
    s = lax.axis_index("s")
    w = s * _NC + c

    def outer(g, carry):
        base0 = w * _EPW + g * (_GNBUF * _K)
        for b in range(_GNBUF):
            @pl.when(g > 0)
            def _(b=b):
                pltpu.make_async_copy(bufr[b], out_d.at[pl.ds(base0, _K)], osem[b]).wait()
            pltpu.async_copy(send.at[pl.ds(base0 + b * _K, _K)], sidx[b], isem[b])
            pltpu.async_copy(recv.at[pl.ds(base0 + b * _K, _K)], ridx[b], rsem[b])
        for b in range(_GNBUF):
            pltpu.make_async_copy(send.at[pl.ds(base0 + b * _K, _K)], sidx[b], isem[b]).wait()
            pltpu.make_async_copy(recv.at[pl.ds(base0 + b * _K, _K)], ridx[b], rsem[b]).wait()
            pltpu.async_copy(pos16.at[sidx[b]], bufs[b], gssem[b])
            pltpu.async_copy(pos16.at[ridx[b]], bufr[b], grsem[b])
        for b in range(_GNBUF):
            pltpu.make_async_copy(pos16.at[sidx[b]], bufs[b], gssem[b]).wait()
            pltpu.make_async_copy(pos16.at[ridx[b]], bufr[b], grsem[b]).wait()

            def sub_row(k, cr, b=b):
                bufr[b][k, :] = bufr[b][k, :] - bufs[b][k, :]
                return cr

            lax.fori_loop(0, _K, sub_row, 0)
            pltpu.async_copy(bufr[b], out_d.at[pl.ds(base0 + b * _K, _K)], osem[b])
        return carry

    lax.fori_loop(0, _GOUTER, outer, 0)
    for b in range(_GNBUF):
        pltpu.make_async_copy(bufr[b], out_d.at[pl.ds(0, _K)], osem[b]).wait()


_NBUF = 4
_OUTER = _CHUNKS // _NBUF


def _msg_body(hu, coeff, send, recv, zrows, out, *scr):
    sidx = scr[0:_NBUF]
    ridx = scr[_NBUF:2 * _NBUF]
    cbuf = scr[2 * _NBUF:3 * _NBUF]
    rows = scr[3 * _NBUF:4 * _NBUF]
    acc = scr[4 * _NBUF]
    isem = scr[4 * _NBUF + 1:5 * _NBUF + 1]
    rsem = scr[5 * _NBUF + 1:6 * _NBUF + 1]
    gsem = scr[6 * _NBUF + 1:7 * _NBUF + 1]
    csem = scr[7 * _NBUF + 1:8 * _NBUF + 1]
    ssem = scr[8 * _NBUF + 1:9 * _NBUF + 1]
    c = lax.axis_index("c")
    s = lax.axis_index("s")
    pltpu.sync_copy(zrows, acc.at[pl.ds(s * _RPT, _RPT)])
    plsc.subcore_barrier()

    def outer(g, carry):
        base0 = s * _EPT + g * (_NBUF * _K)
        for b in range(_NBUF):
            @pl.when(g > 0)
            def _(b=b):
                pltpu.make_async_copy(rows[b], acc.at[ridx[b]], ssem[b]).wait()
            pltpu.async_copy(send.at[pl.ds(base0 + b * _K, _K)], sidx[b], isem[b])
            pltpu.async_copy(recv.at[pl.ds(base0 + b * _K, _K)], ridx[b], rsem[b])
        for b in range(_NBUF):
            pltpu.make_async_copy(send.at[pl.ds(base0 + b * _K, _K)], sidx[b], isem[b]).wait()
            pltpu.make_async_copy(recv.at[pl.ds(base0 + b * _K, _K)], ridx[b], rsem[b]).wait()
            pltpu.async_copy(hu.at[sidx[b]], rows[b], gsem[b])
            pltpu.async_copy(coeff.at[c, pl.ds(base0 + b * _K, _K)], cbuf[b], csem[b])
        for b in range(_NBUF):
            pltpu.make_async_copy(hu.at[sidx[b]], rows[b], gsem[b]).wait()
            pltpu.make_async_copy(coeff.at[c, pl.ds(base0 + b * _K, _K)], cbuf[b], csem[b]).wait()

            def mul_row(k, cr, b=b):
                for j in range(_D // 16):
                    sl = pl.ds(j * 16, 16)
                    rows[b][k, sl] = rows[b][k, sl] * cbuf[b][k, sl]
                return cr

            lax.fori_loop(0, _K, mul_row, 0)
            pltpu.async_copy(rows[b], acc.at[ridx[b]], ssem[b], add=True)
        return carry

    lax.fori_loop(0, _OUTER, outer, 0)
    for b in range(_NBUF):
        pltpu.make_async_copy(rows[b], acc.at[ridx[b]], ssem[b]).wait()
    plsc.subcore_barrier()
    pltpu.sync_copy(acc.at[pl.ds(s * _RPT, _RPT)],
                    out.at[c, pl.ds(s * _RPT, _RPT)])


@functools.cache
def _sc_kernels():
    mesh = plsc.VectorSubcoreMesh(core_axis_name="c", subcore_axis_name="s")
    pos_gather = pl.kernel(
        _pos_gather_body,
        mesh=mesh,
        compiler_params=pltpu.CompilerParams(use_tc_tiling_on_sc=False),
        out_type=jax.ShapeDtypeStruct((_E, 16), jnp.float32),
        scratch_types=(
            [pltpu.VMEM((_K,), jnp.int32)] * (2 * _GNBUF)
            + [pltpu.VMEM((_K, 16), jnp.float32)] * (2 * _GNBUF)
            + [pltpu.SemaphoreType.DMA] * (5 * _GNBUF)
        ),
    )
    msg_pass = pl.kernel(
        _msg_body,
        mesh=mesh,
        compiler_params=pltpu.CompilerParams(use_tc_tiling_on_sc=False),
        out_type=jax.ShapeDtypeStruct((2, _N, _D), jnp.float32),
        scratch_types=(
            [pltpu.VMEM((_K,), jnp.int32)] * (2 * _NBUF)
            + [pltpu.VMEM((_K, _D), jnp.float32)] * (2 * _NBUF)
            + [pltpu.VMEM_SHARED((_N, _D), jnp.float32)]
            + [pltpu.SemaphoreType.DMA] * (5 * _NBUF)
        ),
    )
    return pos_gather, msg_pass


_EPAD = 327680
_EP = _EPAD // 128
_BR = 32
_BV = 512


def _vtr_body(d_ref, ident_ref, vt_ref):
    vt_ref[...] = lax.dot_general(d_ref[...], ident_ref[...],
                                  (((0,), (0,)), ((), ())),
                                  precision=lax.Precision.HIGHEST,
                                  preferred_element_type=jnp.float32)


_vtr_call = pl.pallas_call(
    _vtr_body,
    grid=(_E // _BV,),
    in_specs=[
        pl.BlockSpec((_BV, 16), lambda e: (e, 0)),
        pl.BlockSpec((_BV, _BV), lambda e: (0, 0)),
    ],
    out_specs=pl.BlockSpec((16, _BV), lambda e: (0, e)),
    out_shape=jax.ShapeDtypeStruct((16, _EPAD), jnp.float32),
)


def _geom_body(vx_ref, vy_ref, vz_ref, rb_ref, sh_ref):
    x = vx_ref[0]
    y = vy_ref[0]
    z = vz_ref[0]
    sq = x * x + y * y + z * z
    is0 = sq == 0.0
    r = jnp.sqrt(jnp.where(is0, 1.0, sq))
    r = jnp.where(is0, 0.0, r)
    inv = 1.0 / jnp.where(is0, 1.0, r)
    ux = x * inv
    uy = y * inv
    uz = z * inv
    s3 = math.sqrt(3.0)
    s15 = math.sqrt(15.0)
    sh_ref[0] = s3 * ux
    sh_ref[1] = s3 * uy
    sh_ref[2] = s3 * uz
    sh_ref[3] = s15 * ux * uy
    sh_ref[4] = s15 * uy * uz
    sh_ref[5] = (math.sqrt(5.0) / 2.0) * (3.0 * uz * uz - 1.0)
    sh_ref[6] = s15 * ux * uz
    sh_ref[7] = (s15 / 2.0) * (ux * ux - uy * uy)
    sh_ref[8] = (math.sqrt(70.0) / 4.0) * uy * (3.0 * ux * ux - uy * uy)
    sh_ref[9] = math.sqrt(105.0) * ux * uy * uz
    sh_ref[10] = (math.sqrt(42.0) / 4.0) * uy * (5.0 * uz * uz - 1.0)
    sh_ref[11] = (math.sqrt(7.0) / 2.0) * uz * (5.0 * uz * uz - 3.0)
    sh_ref[12] = (math.sqrt(42.0) / 4.0) * ux * (5.0 * uz * uz - 1.0)
    sh_ref[13] = (math.sqrt(105.0) / 2.0) * uz * (ux * ux - uy * uy)
    sh_ref[14] = (math.sqrt(70.0) / 4.0) * ux * (ux * ux - 3.0 * uy * uy)
    r2 = r * r
    r5 = r2 * r2 * r
    poly = 1.0 - 21.0 * r5 + 35.0 * r5 * r - 15.0 * r5 * r2
    env = jnp.where(r < 1.0, poly, 0.0)
    fac = math.sqrt(2.0) * jnp.where(is0, 0.0, env * inv)
    for k in range(_NB):
        rb_ref[k] = jnp.sin((math.pi * (k + 1)) * r) * fac


_geom_call = pl.pallas_call(
    _geom_body,
    grid=(_EP // _BR,),
    in_specs=[
        pl.BlockSpec((1, _BR, 128), lambda i: (0, i, 0)),
        pl.BlockSpec((1, _BR, 128), lambda i: (1, i, 0)),
        pl.BlockSpec((1, _BR, 128), lambda i: (2, i, 0)),
    ],
    out_specs=[
        pl.BlockSpec((_NB, _BR, 128), lambda i: (0, i, 0)),
        pl.BlockSpec((_SH, _BR, 128), lambda i: (0, i, 0)),
    ],
    out_shape=[
        jax.ShapeDtypeStruct((_NB, _EP, 128), jnp.float32),
        jax.ShapeDtypeStruct((_SH, _EP, 128), jnp.float32),
    ],
)

_BE = 4096


def _coeff_body(rb_ref, sh_ref, m1t_ref, m2t_ref, m3_ref, wsh_ref, out_ref):
    t = _swish(jnp.dot(m1t_ref[...], rb_ref[...],
                       preferred_element_type=jnp.float32))
    t = _swish(jnp.dot(m2t_ref[...], t, preferred_element_type=jnp.float32))
    mix = lax.dot_general(t, m3_ref[...], (((0,), (0,)), ((), ())),
                          preferred_element_type=jnp.float32)
    wsh = lax.dot_general(sh_ref[...], wsh_ref[...], (((0,), (0,)), ((), ())),
                          preferred_element_type=jnp.float32)
    out_ref[0] = mix[:, :_D]
    out_ref[1] = wsh * mix[:, _D:]


_coeff_call = pl.pallas_call(
    _coeff_body,
    grid=(_EPAD // _BE,),
    in_specs=[
        pl.BlockSpec((_NB, _BE), lambda e: (0, e)),
        pl.BlockSpec((_SH, _BE), lambda e: (0, e)),
        pl.BlockSpec((_HID, _NB), lambda e: (0, 0)),
        pl.BlockSpec((_HID, _HID), lambda e: (0, 0)),
        pl.BlockSpec((_HID, 2 * _D), lambda e: (0, 0)),
        pl.BlockSpec((_SH, _D), lambda e: (0, 0)),
    ],
    out_specs=pl.BlockSpec((2, _BE, _D), lambda e: (0, e, 0)),
    out_shape=jax.ShapeDtypeStruct((2, _EPAD, _D), jnp.float32),
)

_BN = 2000


def _node_body(h_ref, w_ref, sc_ref, hu_ref):
    hw = jnp.dot(h_ref[...], w_ref[...], preferred_element_type=jnp.float32)
    sc_ref[...] = hw[:, :_D]
    hu_ref[...] = hw[:, _D:]


_node_call = pl.pallas_call(
    _node_body,
    grid=(_N // _BN,),
    in_specs=[
        pl.BlockSpec((_BN, _D), lambda i: (i, 0)),
        pl.BlockSpec((_D, 2 * _D), lambda i: (0, 0)),
    ],
    out_specs=[
        pl.BlockSpec((_BN, _D), lambda i: (i, 0)),
        pl.BlockSpec((_BN, _D), lambda i: (i, 0)),
    ],
    out_shape=[
        jax.ShapeDtypeStruct((_N, _D), jnp.float32),
        jax.ShapeDtypeStruct((_N, _D), jnp.float32),
    ],
)


def _down_body(a_ref, b_ref, sc_ref, wt_ref, wb_ref, h_ref):
    t = (jnp.dot(a_ref[0], wt_ref[...], preferred_element_type=jnp.float32)
         + jnp.dot(b_ref[0], wb_ref[...], preferred_element_type=jnp.float32)
         + sc_ref[...])
    h_ref[...] = _swish(t)


_down_call = pl.pallas_call(
    _down_body,
    grid=(_N // _BN,),
    in_specs=[
        pl.BlockSpec((1, _BN, _D), lambda i: (0, i, 0)),
        pl.BlockSpec((1, _BN, _D), lambda i: (1, i, 0)),
        pl.BlockSpec((_BN, _D), lambda i: (i, 0)),
        pl.BlockSpec((_D, _D), lambda i: (0, 0)),
        pl.BlockSpec((_D, _D), lambda i: (0, 0)),
    ],
    out_specs=pl.BlockSpec((_BN, _D), lambda i: (i, 0)),
    out_shape=jax.ShapeDtypeStruct((_N, _D), jnp.float32),
)


def kernel(positions, node_features, senders, receivers, W_sc, W_up, W_sh, W_down, M1, M2, M3):
    pos_gather, msg_pass = _sc_kernels()
    pos16 = jnp.pad(positions, ((0, 0), (0, 13)))
    dvec = pos_gather(pos16, senders, receivers)
    vt3 = _vtr_call(dvec, jnp.eye(_BV, dtype=jnp.float32)).reshape(16, _EP, 128)
    rb3, sh3 = _geom_call(vt3, vt3, vt3)
    rb2 = rb3.reshape(_NB, _EPAD)
    sh2 = sh3.reshape(_SH, _EPAD)
    coeffs = [
        _coeff_call(rb2, sh2, M1[l].T, M2[l].T, M3[l], W_sh[l])
        for l in range(_NL)
    ]
    zrows = jnp.zeros((_RPT, _D), jnp.float32)
    h = node_features
    for l in range(_NL):
        wn = jnp.concatenate([W_sc[l], W_up[l]], axis=1)
        sc, hu = _node_call(h, wn)
        agg = msg_pass(hu, coeffs[l], senders, receivers, zrows)
        h = _down_call(agg, agg, sc, W_down[l, :_D], W_down[l, _D:])
    return h[:, :3]
